# baseline (speedup 1.0000x reference)
import functools

import jax
import jax.numpy as jnp
from jax import lax
from jax.experimental import pallas as pl
from jax.experimental.pallas import tpu as pltpu
from jax.experimental.pallas import tpu_sc as plsc


def _adj_build_kernel(mapt_hbm, a2bt_hbm, adjb_hbm, adja_hbm,
                      idx_v, idxa_v, adj_v, adja_v, sem,
                      *, nb, na, maxb, mols_per_worker, num_cores):
    L = 16
    wid = lax.axis_index("s") * num_cores + lax.axis_index("c")
    ones = jnp.full((L,), 1.0, dtype=jnp.float32)
    lane = lax.iota(jnp.int32, L)

    def per_mol(mol, _):
        b = wid * mols_per_worker + mol
        pltpu.sync_copy(mapt_hbm.at[b], idx_v)
        pltpu.sync_copy(a2bt_hbm.at[b], idxa_v)

        def zrow(r, _):
            for c in range(nb // L):
                adj_v[pl.ds(r * nb + c * L, L)] = jnp.zeros((L,), jnp.float32)
            return ()
        lax.fori_loop(0, nb, zrow, ())

        def zrow_a(r, _):
            for c in range(nb // L):
                adja_v[pl.ds(r * nb + c * L, L)] = jnp.zeros((L,),
                                                            jnp.float32)
            return ()
        lax.fori_loop(0, na, zrow_a, ())

        for k in range(maxb):
            for c in range(nb // L):
                rows = lane + c * L
                cols = idx_v[pl.ds(k * nb + c * L, L)]
                plsc.addupdate_scatter(adj_v, [rows * nb + cols], ones)
        for k in range(maxb):
            for c in range(na // L):
                rows = lane + c * L
                cols = idxa_v[pl.ds(k * na + c * L, L)]
                plsc.addupdate_scatter(adja_v, [rows * nb + cols], ones)

        pltpu.sync_copy(adj_v, adjb_hbm.at[pl.ds(b * nb * nb, nb * nb)])
        pltpu.sync_copy(adja_v, adja_hbm.at[pl.ds(b * na * nb, na * nb)])
        return ()

    lax.fori_loop(0, mols_per_worker, per_mol, ())


def _build_adjacency(mapping, a2b, B, NB, NA, MAXB):
    mapt = mapping.transpose(0, 2, 1).reshape(B, MAXB * NB)
    a2bt = a2b.transpose(0, 2, 1).reshape(B, MAXB * NA)

    info = plsc.get_sparse_core_info()
    NW = info.num_cores * info.num_subcores
    mols_per_worker = B // NW

    mesh = plsc.VectorSubcoreMesh(core_axis_name="c", subcore_axis_name="s")
    body = functools.partial(
        _adj_build_kernel, nb=NB, na=NA, maxb=MAXB,
        mols_per_worker=mols_per_worker, num_cores=info.num_cores)
    return pl.kernel(
        body,
        mesh=mesh,
        compiler_params=pltpu.CompilerParams(needs_layout_passes=False),
        out_type=[
            jax.ShapeDtypeStruct((B * NB * NB,), jnp.float32),
            jax.ShapeDtypeStruct((B * NA * NB,), jnp.float32),
        ],
        scratch_types=[
            pltpu.VMEM((MAXB * NB,), jnp.int32),
            pltpu.VMEM((MAXB * NA,), jnp.int32),
            pltpu.VMEM((NB * NB,), jnp.float32),
            pltpu.VMEM((NA * NB,), jnp.float32),
            pltpu.SemaphoreType.DMA,
        ],
    )(mapt, a2bt)


def _dmpnn_kernel(ml_ref, f_ref, adjb_ref, adja_ref, atom_ref, wi_ref,
                  wh_ref, woa_ref, wob_ref, bo_ref, comp_ref, mask_ref,
                  *, nb, na, comp_dim, mols):
    f32 = jnp.float32

    for i in range(mols):
        f = f_ref[i]
        inp = jnp.dot(f, wi_ref[...], preferred_element_type=f32)
        msg = jnp.maximum(inp, 0.0)

        adj = adjb_ref[pl.ds(i * nb, nb), :]
        for _ in range(2):
            agg = jnp.dot(adj, msg, preferred_element_type=f32)
            msg = jnp.maximum(inp + jnp.dot(agg, wh_ref[...],
                                            preferred_element_type=f32), 0.0)

        adj_a = adja_ref[pl.ds(i * na, na), :]
        atom_msg = jnp.dot(adj_a, msg, preferred_element_type=f32)

        hidden = jnp.dot(atom_ref[i], woa_ref[...],
                         preferred_element_type=f32)
        hidden = hidden + jnp.dot(atom_msg, wob_ref[...],
                                  preferred_element_type=f32)
        hidden = jnp.maximum(hidden + bo_ref[...], 0.0)
        comp_ref[pl.ds(i * na, na), :] = hidden

    mask_rows = (jax.lax.broadcasted_iota(jnp.int32, (mols, 1, comp_dim), 2)
                 < ml_ref[0]).astype(f32)
    mask_ref[...] = mask_rows


def kernel(atom_features, f_ini_atoms_bonds, atom_to_incoming_bonds, mapping,
           global_features, W_i, W_h, W_o, b_o, mol_len):
    B, NA, AF = atom_features.shape
    _, NB, AFBF = f_ini_atoms_bonds.shape
    H = W_i.shape[1]
    MAXB = mapping.shape[2]
    comp_dim = max(NA, H)
    assert comp_dim == H

    ml = jnp.asarray(mol_len, jnp.int32).reshape(1)
    mapping = mapping.astype(jnp.int32)
    a2b = atom_to_incoming_bonds.astype(jnp.int32)
    W_oa = W_o[:AF]
    W_ob = W_o[AF:]
    b_o2 = b_o.reshape(1, H)

    adjb, adja = _build_adjacency(mapping, a2b, B, NB, NA, MAXB)
    adjb = adjb.reshape(B * NB, NB)
    adja = adja.reshape(B * NA, NB)

    MOLS = 4
    body = functools.partial(_dmpnn_kernel, nb=NB, na=NA,
                             comp_dim=comp_dim, mols=MOLS)

    comp, c_mask = pl.pallas_call(
        body,
        grid=(B // MOLS,),
        in_specs=[
            pl.BlockSpec(memory_space=pltpu.SMEM),
            pl.BlockSpec((MOLS, NB, AFBF), lambda b: (b, 0, 0)),
            pl.BlockSpec((MOLS * NB, NB), lambda b: (b, 0)),
            pl.BlockSpec((MOLS * NA, NB), lambda b: (b, 0)),
            pl.BlockSpec((MOLS, NA, AF), lambda b: (b, 0, 0)),
            pl.BlockSpec((AFBF, H), lambda b: (0, 0)),
            pl.BlockSpec((H, H), lambda b: (0, 0)),
            pl.BlockSpec((AF, H), lambda b: (0, 0)),
            pl.BlockSpec((H, H), lambda b: (0, 0)),
            pl.BlockSpec((1, H), lambda b: (0, 0)),
        ],
        out_specs=[
            pl.BlockSpec((MOLS * NA, comp_dim), lambda b: (b, 0)),
            pl.BlockSpec((MOLS, 1, comp_dim), lambda b: (b, 0, 0)),
        ],
        out_shape=[
            jax.ShapeDtypeStruct((B * NA, comp_dim), jnp.float32),
            jax.ShapeDtypeStruct((B, 1, comp_dim), jnp.float32),
        ],
    )(ml, f_ini_atoms_bonds, adjb, adja, atom_features, W_i, W_h, W_oa,
      W_ob, b_o2)
    return comp, c_mask.reshape(B, comp_dim)


# speedup vs baseline: 1.2282x; 1.2282x over previous
import functools

import jax
import jax.numpy as jnp
from jax.experimental import pallas as pl
from jax.experimental.pallas import tpu as pltpu


def _dmpnn_kernel(ml_ref, f_ref, map_ref, a2b_ref, atom_ref, wi_ref, wh_ref,
                  woa_ref, wob_ref, bo_ref, comp_ref, mask_ref,
                  *, nb, na, maxb, comp_dim, mols):
    f32 = jnp.float32
    bf16 = jnp.bfloat16
    iota_b = jax.lax.broadcasted_iota(jnp.int32, (nb, nb), 1)
    iota_a = jax.lax.broadcasted_iota(jnp.int32, (na, nb), 1)

    for i in range(mols):
        f = f_ref[i]
        inp = jnp.dot(f, wi_ref[...], preferred_element_type=f32)
        msg = jnp.maximum(inp, 0.0)

        m = map_ref[i]
        adj = jnp.zeros((nb, nb), dtype=bf16)
        for k in range(maxb):
            adj = adj + (m[:, k][:, None] == iota_b).astype(bf16)

        for _ in range(2):
            agg = jnp.dot(adj, msg.astype(bf16), preferred_element_type=f32)
            msg = jnp.maximum(
                inp + jnp.dot(agg.astype(bf16), wh_ref[...],
                              preferred_element_type=f32), 0.0)

        a = a2b_ref[i]
        adj_a = jnp.zeros((na, nb), dtype=bf16)
        for k in range(maxb):
            adj_a = adj_a + (a[:, k][:, None] == iota_a).astype(bf16)
        atom_msg = jnp.dot(adj_a, msg.astype(bf16),
                           preferred_element_type=f32)

        hidden = jnp.dot(atom_ref[i], woa_ref[...],
                         preferred_element_type=f32)
        hidden = hidden + jnp.dot(atom_msg.astype(bf16), wob_ref[...],
                                  preferred_element_type=f32)
        hidden = jnp.maximum(hidden + bo_ref[...], 0.0)
        comp_ref[pl.ds(i * na, na), :] = hidden

    mask_rows = (jax.lax.broadcasted_iota(jnp.int32, (mols, 1, comp_dim), 2)
                 < ml_ref[0]).astype(f32)
    mask_ref[...] = mask_rows


def kernel(atom_features, f_ini_atoms_bonds, atom_to_incoming_bonds, mapping,
           global_features, W_i, W_h, W_o, b_o, mol_len):
    B, NA, AF = atom_features.shape
    _, NB, AFBF = f_ini_atoms_bonds.shape
    H = W_i.shape[1]
    MAXB = mapping.shape[2]
    comp_dim = max(NA, H)
    assert comp_dim == H

    ml = jnp.asarray(mol_len, jnp.int32).reshape(1)
    mapping = mapping.astype(jnp.int32)
    a2b = atom_to_incoming_bonds.astype(jnp.int32)
    bf16 = jnp.bfloat16
    f_bf = f_ini_atoms_bonds.astype(bf16)
    atom_bf = atom_features.astype(bf16)
    W_i = W_i.astype(bf16)
    W_h = W_h.astype(bf16)
    W_oa = W_o[:AF].astype(bf16)
    W_ob = W_o[AF:].astype(bf16)
    b_o2 = b_o.reshape(1, H)

    MOLS = 4
    body = functools.partial(_dmpnn_kernel, nb=NB, na=NA, maxb=MAXB,
                             comp_dim=comp_dim, mols=MOLS)

    comp, c_mask = pl.pallas_call(
        body,
        grid=(B // MOLS,),
        in_specs=[
            pl.BlockSpec(memory_space=pltpu.SMEM),
            pl.BlockSpec((MOLS, NB, AFBF), lambda b: (b, 0, 0)),
            pl.BlockSpec((MOLS, NB, MAXB), lambda b: (b, 0, 0)),
            pl.BlockSpec((MOLS, NA, MAXB), lambda b: (b, 0, 0)),
            pl.BlockSpec((MOLS, NA, AF), lambda b: (b, 0, 0)),
            pl.BlockSpec((AFBF, H), lambda b: (0, 0)),
            pl.BlockSpec((H, H), lambda b: (0, 0)),
            pl.BlockSpec((AF, H), lambda b: (0, 0)),
            pl.BlockSpec((H, H), lambda b: (0, 0)),
            pl.BlockSpec((1, H), lambda b: (0, 0)),
        ],
        out_specs=[
            pl.BlockSpec((MOLS * NA, comp_dim), lambda b: (b, 0)),
            pl.BlockSpec((MOLS, 1, comp_dim), lambda b: (b, 0, 0)),
        ],
        out_shape=[
            jax.ShapeDtypeStruct((B * NA, comp_dim), jnp.float32),
            jax.ShapeDtypeStruct((B, 1, comp_dim), jnp.float32),
        ],
    )(ml, f_bf, mapping, a2b, atom_bf, W_i, W_h, W_oa, W_ob, b_o2)
    return comp, c_mask.reshape(B, comp_dim)


# speedup vs baseline: 1.3192x; 1.0741x over previous
import functools

import jax
import jax.numpy as jnp
from jax.experimental import pallas as pl
from jax.experimental.pallas import tpu as pltpu


def _dmpnn_kernel(ml_ref, f_ref, map_ref, a2b_ref, atom_ref, wi_ref, wh_ref,
                  woa_ref, wob_ref, bo_ref, comp_ref, mask_ref,
                  *, nb, na, maxb, comp_dim, mols):
    f32 = jnp.float32
    iota_b = jax.lax.broadcasted_iota(jnp.int32, (nb, nb), 1)
    iota_a = jax.lax.broadcasted_iota(jnp.int32, (na, nb), 1)

    for i in range(mols):
        f = f_ref[i]
        inp = jnp.dot(f, wi_ref[...], preferred_element_type=f32)
        msg = jnp.maximum(inp, 0.0)

        m = map_ref[i]
        adj = jnp.zeros((nb, nb), dtype=f32)
        for k in range(maxb):
            adj = adj + (m[:, k][:, None] == iota_b).astype(f32)

        for _ in range(2):
            agg = jnp.dot(adj, msg, preferred_element_type=f32)
            msg = jnp.maximum(inp + jnp.dot(agg, wh_ref[...],
                                            preferred_element_type=f32), 0.0)

        a = a2b_ref[i]
        adj_a = jnp.zeros((na, nb), dtype=f32)
        for k in range(maxb):
            adj_a = adj_a + (a[:, k][:, None] == iota_a).astype(f32)
        atom_msg = jnp.dot(adj_a, msg, preferred_element_type=f32)

        hidden = jnp.dot(atom_ref[i], woa_ref[...],
                         preferred_element_type=f32)
        hidden = hidden + jnp.dot(atom_msg, wob_ref[...],
                                  preferred_element_type=f32)
        hidden = jnp.maximum(hidden + bo_ref[...], 0.0)
        comp_ref[pl.ds(i * na, na), :] = hidden

    mask_rows = (jax.lax.broadcasted_iota(jnp.int32, (mols, 1, comp_dim), 2)
                 < ml_ref[0]).astype(f32)
    mask_ref[...] = mask_rows


def kernel(atom_features, f_ini_atoms_bonds, atom_to_incoming_bonds, mapping,
           global_features, W_i, W_h, W_o, b_o, mol_len):
    B, NA, AF = atom_features.shape
    _, NB, AFBF = f_ini_atoms_bonds.shape
    H = W_i.shape[1]
    MAXB = mapping.shape[2]
    comp_dim = max(NA, H)
    assert comp_dim == H

    ml = jnp.asarray(mol_len, jnp.int32).reshape(1)
    mapping = mapping.astype(jnp.int32)
    a2b = atom_to_incoming_bonds.astype(jnp.int32)
    W_oa = W_o[:AF]
    W_ob = W_o[AF:]
    b_o2 = b_o.reshape(1, H)

    MOLS = 8
    body = functools.partial(_dmpnn_kernel, nb=NB, na=NA, maxb=MAXB,
                             comp_dim=comp_dim, mols=MOLS)

    comp, c_mask = pl.pallas_call(
        body,
        grid=(B // MOLS,),
        in_specs=[
            pl.BlockSpec(memory_space=pltpu.SMEM),
            pl.BlockSpec((MOLS, NB, AFBF), lambda b: (b, 0, 0)),
            pl.BlockSpec((MOLS, NB, MAXB), lambda b: (b, 0, 0)),
            pl.BlockSpec((MOLS, NA, MAXB), lambda b: (b, 0, 0)),
            pl.BlockSpec((MOLS, NA, AF), lambda b: (b, 0, 0)),
            pl.BlockSpec((AFBF, H), lambda b: (0, 0)),
            pl.BlockSpec((H, H), lambda b: (0, 0)),
            pl.BlockSpec((AF, H), lambda b: (0, 0)),
            pl.BlockSpec((H, H), lambda b: (0, 0)),
            pl.BlockSpec((1, H), lambda b: (0, 0)),
        ],
        out_specs=[
            pl.BlockSpec((MOLS * NA, comp_dim), lambda b: (b, 0)),
            pl.BlockSpec((MOLS, 1, comp_dim), lambda b: (b, 0, 0)),
        ],
        out_shape=[
            jax.ShapeDtypeStruct((B * NA, comp_dim), jnp.float32),
            jax.ShapeDtypeStruct((B, 1, comp_dim), jnp.float32),
        ],
    )(ml, f_ini_atoms_bonds, mapping, a2b, atom_features, W_i, W_h, W_oa,
      W_ob, b_o2)
    return comp, c_mask.reshape(B, comp_dim)
